# baseline (speedup 1.0000x reference)
import functools

import jax
import jax.numpy as jnp
from jax import lax
from jax.experimental import pallas as pl
from jax.experimental.pallas import tpu as pltpu
from jax.experimental.pallas import tpu_sc as plsc

_NC = 2
_NS = 16
_NW = _NC * _NS
_K = 128
_SC0_FRAC = 0.34
_F = 128
_ROW_BLOCK = 2000


def _segment_sum_sc(h, src_p, dst_p, zeros, n_acc, ch0, ch1):
    rpt = n_acc // _NS
    chmax = max(ch0, ch1)
    mesh = plsc.VectorSubcoreMesh(core_axis_name="c", subcore_axis_name="s")

    @functools.partial(
        pl.kernel,
        mesh=mesh,
        out_type=jax.ShapeDtypeStruct((_NC, n_acc, _F), jnp.float32),
        scratch_types=[
            pltpu.VMEM((chmax, _K), jnp.int32),
            pltpu.VMEM((chmax, _K), jnp.int32),
            pltpu.VMEM((_K, _F), jnp.float32),
            pltpu.VMEM_SHARED((n_acc, _F), jnp.float32),
            pltpu.SemaphoreType.DMA,
        ],
    )
    def seg(h_hbm, src_hbm, dst_hbm, z_hbm, out_hbm, src_v, dst_v, rows,
            acc, sem):
        c = lax.axis_index("c")
        s = lax.axis_index("s")
        pltpu.sync_copy(z_hbm, acc.at[pl.ds(s * rpt, rpt)])
        pltpu.sync_copy(src_hbm.at[c, s], src_v)
        pltpu.sync_copy(dst_hbm.at[c, s], dst_v)
        plsc.subcore_barrier()

        def body(j, carry):
            pltpu.async_copy(h_hbm.at[src_v.at[j]], rows, sem).wait()
            pltpu.sync_copy(rows, acc.at[dst_v.at[j]], add=True)
            return carry

        nch = jnp.where(c == 0, ch0, ch1)
        lax.fori_loop(0, nch, body, 0)
        plsc.subcore_barrier()
        pltpu.sync_copy(acc.at[pl.ds(s * rpt, rpt)],
                        out_hbm.at[c, pl.ds(s * rpt, rpt)])

    return seg(h, src_p, dst_p, zeros)


def _dense_first(x, W, b):
    n, f_in = x.shape
    h = W.shape[1]

    def body(x_ref, w_ref, b_ref, o_ref):
        o_ref[...] = jnp.maximum(
            jnp.dot(x_ref[...], w_ref[...], preferred_element_type=jnp.float32)
            + b_ref[...], 0.0)

    return pl.pallas_call(
        body,
        grid=(n // _ROW_BLOCK,),
        in_specs=[
            pl.BlockSpec((_ROW_BLOCK, f_in), lambda i: (i, 0)),
            pl.BlockSpec((f_in, h), lambda i: (0, 0)),
            pl.BlockSpec((1, h), lambda i: (0, 0)),
        ],
        out_specs=pl.BlockSpec((_ROW_BLOCK, h), lambda i: (i, 0)),
        out_shape=jax.ShapeDtypeStruct((n, h), jnp.float32),
    )(x, W, b.reshape(1, -1))


def _dense_mid(h, parts, W, b, fw):
    n, f = h.shape

    def body(h_ref, p_ref, w_ref, b_ref, fw_ref, o_ref):
        hh = h_ref[...]
        t = hh + p_ref[0] + p_ref[1]
        o_ref[...] = jnp.maximum(
            jnp.dot(t, w_ref[...], preferred_element_type=jnp.float32)
            + b_ref[...], 0.0) + fw_ref[0, 0] * hh

    return pl.pallas_call(
        body,
        grid=(n // _ROW_BLOCK,),
        in_specs=[
            pl.BlockSpec((_ROW_BLOCK, f), lambda i: (i, 0)),
            pl.BlockSpec((2, _ROW_BLOCK, f), lambda i: (0, i, 0)),
            pl.BlockSpec((f, f), lambda i: (0, 0)),
            pl.BlockSpec((1, f), lambda i: (0, 0)),
            pl.BlockSpec((1, 1), lambda i: (0, 0)),
        ],
        out_specs=pl.BlockSpec((_ROW_BLOCK, f), lambda i: (i, 0)),
        out_shape=jax.ShapeDtypeStruct((n, f), jnp.float32),
    )(h, parts, W, b.reshape(1, -1), fw.reshape(1, 1))


def _dense_final(h, parts, W2, b2, fw, Wo, bo):
    n, f = h.shape
    c_dim = Wo.shape[1]

    def body(h_ref, p_ref, w2_ref, b2_ref, fw_ref, wo_ref, bo_ref, o_ref):
        hh = h_ref[...]
        t = hh + p_ref[0] + p_ref[1]
        g = jnp.maximum(
            jnp.dot(t, w2_ref[...], preferred_element_type=jnp.float32)
            + b2_ref[...], 0.0) + fw_ref[0, 0] * hh
        logits = jnp.dot(g, wo_ref[...], preferred_element_type=jnp.float32) + bo_ref[...]
        m = jnp.max(logits, axis=-1, keepdims=True)
        lse = jnp.log(jnp.sum(jnp.exp(logits - m), axis=-1, keepdims=True)) + m
        o_ref[...] = logits - lse

    return pl.pallas_call(
        body,
        grid=(n // _ROW_BLOCK,),
        in_specs=[
            pl.BlockSpec((_ROW_BLOCK, f), lambda i: (i, 0)),
            pl.BlockSpec((2, _ROW_BLOCK, f), lambda i: (0, i, 0)),
            pl.BlockSpec((f, f), lambda i: (0, 0)),
            pl.BlockSpec((1, f), lambda i: (0, 0)),
            pl.BlockSpec((1, 1), lambda i: (0, 0)),
            pl.BlockSpec((f, c_dim), lambda i: (0, 0)),
            pl.BlockSpec((1, c_dim), lambda i: (0, 0)),
        ],
        out_specs=pl.BlockSpec((_ROW_BLOCK, c_dim), lambda i: (i, 0)),
        out_shape=jax.ShapeDtypeStruct((n, c_dim), jnp.float32),
    )(h, parts, W2, b2.reshape(1, -1), fw.reshape(1, 1), Wo, bo.reshape(1, -1))


def kernel(x, edge_index, edge_weight, W_first, b_first, W_c1, b_c1, W_c2,
           b_c2, W_out, b_out, fuse_weight):
    n = x.shape[0]
    e = edge_index.shape[1]
    cht = -(-e // (_NS * _K))
    ch0 = max(1, int(round(cht * _SC0_FRAC)))
    ch1 = cht - ch0
    chmax = max(ch0, ch1)
    e_pad = _NS * cht * _K
    n_acc = -(-(n + 1) // (_NS * 8)) * (_NS * 8)

    src = edge_index[0]
    dst = edge_index[1]
    pad = e_pad - e
    src_f = jnp.concatenate([src, jnp.zeros((pad,), src.dtype)])
    dst_f = jnp.concatenate([dst, jnp.full((pad,), n, dst.dtype)])

    def split(a):
        part0 = a[: _NS * ch0 * _K].reshape(_NS, ch0, _K)
        part1 = a[_NS * ch0 * _K:].reshape(_NS, ch1, _K)
        part0 = jnp.pad(part0, ((0, 0), (0, chmax - ch0), (0, 0)))
        part1 = jnp.pad(part1, ((0, 0), (0, chmax - ch1), (0, 0)))
        return jnp.stack([part0, part1])

    src_p, dst_p = split(src_f), split(dst_f)
    zeros = jnp.zeros((n_acc // _NS, _F), jnp.float32)

    h0 = _dense_first(x, W_first, b_first)
    p1 = _segment_sum_sc(h0, src_p, dst_p, zeros, n_acc, ch0, ch1)
    h1 = _dense_mid(h0, p1, W_c1, b_c1, fuse_weight[0])
    p2 = _segment_sum_sc(h1, src_p, dst_p, zeros, n_acc, ch0, ch1)
    return _dense_final(h1, p2, W_c2, b_c2, fuse_weight[1], W_out, b_out)


# speedup vs baseline: 1.0037x; 1.0037x over previous
import functools

import jax
import jax.numpy as jnp
from jax import lax
from jax.experimental import pallas as pl
from jax.experimental.pallas import tpu as pltpu
from jax.experimental.pallas import tpu_sc as plsc

_NC = 2
_NS = 16
_NW = _NC * _NS
_K = 128
_SC0_FRAC = 0.42
_F = 128
_ROW_BLOCK = 2000


def _segment_sum_sc(h, src_p, dst_p, zeros, n_acc, ch0, ch1):
    rpt = n_acc // _NS
    chmax = max(ch0, ch1)
    mesh = plsc.VectorSubcoreMesh(core_axis_name="c", subcore_axis_name="s")

    @functools.partial(
        pl.kernel,
        mesh=mesh,
        out_type=jax.ShapeDtypeStruct((_NC, n_acc, _F), jnp.float32),
        scratch_types=[
            pltpu.VMEM((chmax, _K), jnp.int32),
            pltpu.VMEM((chmax, _K), jnp.int32),
            pltpu.VMEM((_K, _F), jnp.float32),
            pltpu.VMEM_SHARED((n_acc, _F), jnp.float32),
            pltpu.SemaphoreType.DMA,
        ],
    )
    def seg(h_hbm, src_hbm, dst_hbm, z_hbm, out_hbm, src_v, dst_v, rows,
            acc, sem):
        c = lax.axis_index("c")
        s = lax.axis_index("s")
        pltpu.sync_copy(z_hbm, acc.at[pl.ds(s * rpt, rpt)])
        pltpu.sync_copy(src_hbm.at[c, s], src_v)
        pltpu.sync_copy(dst_hbm.at[c, s], dst_v)
        plsc.subcore_barrier()

        def body(j, carry):
            pltpu.async_copy(h_hbm.at[src_v.at[j]], rows, sem).wait()
            pltpu.sync_copy(rows, acc.at[dst_v.at[j]], add=True)
            return carry

        nch = jnp.where(c == 0, ch0, ch1)
        lax.fori_loop(0, nch, body, 0)
        plsc.subcore_barrier()
        pltpu.sync_copy(acc.at[pl.ds(s * rpt, rpt)],
                        out_hbm.at[c, pl.ds(s * rpt, rpt)])

    return seg(h, src_p, dst_p, zeros)


def _dense_first(x, W, b):
    n, f_in = x.shape
    h = W.shape[1]

    def body(x_ref, w_ref, b_ref, o_ref):
        o_ref[...] = jnp.maximum(
            jnp.dot(x_ref[...], w_ref[...], preferred_element_type=jnp.float32)
            + b_ref[...], 0.0)

    return pl.pallas_call(
        body,
        grid=(n // _ROW_BLOCK,),
        in_specs=[
            pl.BlockSpec((_ROW_BLOCK, f_in), lambda i: (i, 0)),
            pl.BlockSpec((f_in, h), lambda i: (0, 0)),
            pl.BlockSpec((1, h), lambda i: (0, 0)),
        ],
        out_specs=pl.BlockSpec((_ROW_BLOCK, h), lambda i: (i, 0)),
        out_shape=jax.ShapeDtypeStruct((n, h), jnp.float32),
    )(x, W, b.reshape(1, -1))


def _dense_mid(h, parts, W, b, fw):
    n, f = h.shape

    def body(h_ref, p_ref, w_ref, b_ref, fw_ref, o_ref):
        hh = h_ref[...]
        t = hh + p_ref[0] + p_ref[1]
        o_ref[...] = jnp.maximum(
            jnp.dot(t, w_ref[...], preferred_element_type=jnp.float32)
            + b_ref[...], 0.0) + fw_ref[0, 0] * hh

    return pl.pallas_call(
        body,
        grid=(n // _ROW_BLOCK,),
        in_specs=[
            pl.BlockSpec((_ROW_BLOCK, f), lambda i: (i, 0)),
            pl.BlockSpec((2, _ROW_BLOCK, f), lambda i: (0, i, 0)),
            pl.BlockSpec((f, f), lambda i: (0, 0)),
            pl.BlockSpec((1, f), lambda i: (0, 0)),
            pl.BlockSpec((1, 1), lambda i: (0, 0)),
        ],
        out_specs=pl.BlockSpec((_ROW_BLOCK, f), lambda i: (i, 0)),
        out_shape=jax.ShapeDtypeStruct((n, f), jnp.float32),
    )(h, parts, W, b.reshape(1, -1), fw.reshape(1, 1))


def _dense_final(h, parts, W2, b2, fw, Wo, bo):
    n, f = h.shape
    c_dim = Wo.shape[1]

    def body(h_ref, p_ref, w2_ref, b2_ref, fw_ref, wo_ref, bo_ref, o_ref):
        hh = h_ref[...]
        t = hh + p_ref[0] + p_ref[1]
        g = jnp.maximum(
            jnp.dot(t, w2_ref[...], preferred_element_type=jnp.float32)
            + b2_ref[...], 0.0) + fw_ref[0, 0] * hh
        logits = jnp.dot(g, wo_ref[...], preferred_element_type=jnp.float32) + bo_ref[...]
        m = jnp.max(logits, axis=-1, keepdims=True)
        lse = jnp.log(jnp.sum(jnp.exp(logits - m), axis=-1, keepdims=True)) + m
        o_ref[...] = logits - lse

    return pl.pallas_call(
        body,
        grid=(n // _ROW_BLOCK,),
        in_specs=[
            pl.BlockSpec((_ROW_BLOCK, f), lambda i: (i, 0)),
            pl.BlockSpec((2, _ROW_BLOCK, f), lambda i: (0, i, 0)),
            pl.BlockSpec((f, f), lambda i: (0, 0)),
            pl.BlockSpec((1, f), lambda i: (0, 0)),
            pl.BlockSpec((1, 1), lambda i: (0, 0)),
            pl.BlockSpec((f, c_dim), lambda i: (0, 0)),
            pl.BlockSpec((1, c_dim), lambda i: (0, 0)),
        ],
        out_specs=pl.BlockSpec((_ROW_BLOCK, c_dim), lambda i: (i, 0)),
        out_shape=jax.ShapeDtypeStruct((n, c_dim), jnp.float32),
    )(h, parts, W2, b2.reshape(1, -1), fw.reshape(1, 1), Wo, bo.reshape(1, -1))


def kernel(x, edge_index, edge_weight, W_first, b_first, W_c1, b_c1, W_c2,
           b_c2, W_out, b_out, fuse_weight):
    n = x.shape[0]
    e = edge_index.shape[1]
    cht = -(-e // (_NS * _K))
    ch0 = max(1, int(round(cht * _SC0_FRAC)))
    ch1 = cht - ch0
    chmax = max(ch0, ch1)
    e_pad = _NS * cht * _K
    n_acc = -(-(n + 1) // (_NS * 8)) * (_NS * 8)

    src = edge_index[0]
    dst = edge_index[1]
    pad = e_pad - e
    src_f = jnp.concatenate([src, jnp.zeros((pad,), src.dtype)])
    dst_f = jnp.concatenate([dst, jnp.full((pad,), n, dst.dtype)])

    def split(a):
        part0 = a[: _NS * ch0 * _K].reshape(_NS, ch0, _K)
        part1 = a[_NS * ch0 * _K:].reshape(_NS, ch1, _K)
        part0 = jnp.pad(part0, ((0, 0), (0, chmax - ch0), (0, 0)))
        part1 = jnp.pad(part1, ((0, 0), (0, chmax - ch1), (0, 0)))
        return jnp.stack([part0, part1])

    src_p, dst_p = split(src_f), split(dst_f)
    zeros = jnp.zeros((n_acc // _NS, _F), jnp.float32)

    h0 = _dense_first(x, W_first, b_first)
    p1 = _segment_sum_sc(h0, src_p, dst_p, zeros, n_acc, ch0, ch1)
    h1 = _dense_mid(h0, p1, W_c1, b_c1, fuse_weight[0])
    p2 = _segment_sum_sc(h1, src_p, dst_p, zeros, n_acc, ch0, ch1)
    return _dense_final(h1, p2, W_c2, b_c2, fuse_weight[1], W_out, b_out)


# speedup vs baseline: 1.0400x; 1.0362x over previous
import functools

import jax
import jax.numpy as jnp
from jax import lax
from jax.experimental import pallas as pl
from jax.experimental.pallas import tpu as pltpu
from jax.experimental.pallas import tpu_sc as plsc

_NC = 2
_NS = 16
_NW = _NC * _NS
_K = 128
_SC0_FRAC = 0.38
_F = 128
_ROW_BLOCK = 2000


def _segment_sum_sc(h, src_p, dst_p, zeros, n_acc, ch0, ch1):
    rpt = n_acc // _NS
    chmax = max(ch0, ch1)
    mesh = plsc.VectorSubcoreMesh(core_axis_name="c", subcore_axis_name="s")

    @functools.partial(
        pl.kernel,
        mesh=mesh,
        out_type=jax.ShapeDtypeStruct((_NC, n_acc, _F), jnp.float32),
        scratch_types=[
            pltpu.VMEM((chmax, _K), jnp.int32),
            pltpu.VMEM((chmax, _K), jnp.int32),
            pltpu.VMEM((_K, _F), jnp.float32),
            pltpu.VMEM_SHARED((n_acc, _F), jnp.float32),
            pltpu.SemaphoreType.DMA,
        ],
    )
    def seg(h_hbm, src_hbm, dst_hbm, z_hbm, out_hbm, src_v, dst_v, rows,
            acc, sem):
        c = lax.axis_index("c")
        s = lax.axis_index("s")
        pltpu.sync_copy(z_hbm, acc.at[pl.ds(s * rpt, rpt)])
        pltpu.sync_copy(src_hbm.at[c, s], src_v)
        pltpu.sync_copy(dst_hbm.at[c, s], dst_v)
        plsc.subcore_barrier()

        def body(j, carry):
            pltpu.async_copy(h_hbm.at[src_v.at[j]], rows, sem).wait()
            pltpu.sync_copy(rows, acc.at[dst_v.at[j]], add=True)
            return carry

        nch = jnp.where(c == 0, ch0, ch1)
        lax.fori_loop(0, nch, body, 0)
        plsc.subcore_barrier()
        pltpu.sync_copy(acc.at[pl.ds(s * rpt, rpt)],
                        out_hbm.at[c, pl.ds(s * rpt, rpt)])

    return seg(h, src_p, dst_p, zeros)


def _dense_first(x, W, b):
    n, f_in = x.shape
    h = W.shape[1]

    def body(x_ref, w_ref, b_ref, o_ref):
        o_ref[...] = jnp.maximum(
            jnp.dot(x_ref[...], w_ref[...], preferred_element_type=jnp.float32)
            + b_ref[...], 0.0)

    return pl.pallas_call(
        body,
        grid=(n // _ROW_BLOCK,),
        in_specs=[
            pl.BlockSpec((_ROW_BLOCK, f_in), lambda i: (i, 0)),
            pl.BlockSpec((f_in, h), lambda i: (0, 0)),
            pl.BlockSpec((1, h), lambda i: (0, 0)),
        ],
        out_specs=pl.BlockSpec((_ROW_BLOCK, h), lambda i: (i, 0)),
        out_shape=jax.ShapeDtypeStruct((n, h), jnp.float32),
    )(x, W, b.reshape(1, -1))


def _dense_mid(h, parts, W, b, fw):
    n, f = h.shape

    def body(h_ref, p_ref, w_ref, b_ref, fw_ref, o_ref):
        hh = h_ref[...]
        t = hh + p_ref[0] + p_ref[1]
        o_ref[...] = jnp.maximum(
            jnp.dot(t, w_ref[...], preferred_element_type=jnp.float32)
            + b_ref[...], 0.0) + fw_ref[0, 0] * hh

    return pl.pallas_call(
        body,
        grid=(n // _ROW_BLOCK,),
        in_specs=[
            pl.BlockSpec((_ROW_BLOCK, f), lambda i: (i, 0)),
            pl.BlockSpec((2, _ROW_BLOCK, f), lambda i: (0, i, 0)),
            pl.BlockSpec((f, f), lambda i: (0, 0)),
            pl.BlockSpec((1, f), lambda i: (0, 0)),
            pl.BlockSpec((1, 1), lambda i: (0, 0)),
        ],
        out_specs=pl.BlockSpec((_ROW_BLOCK, f), lambda i: (i, 0)),
        out_shape=jax.ShapeDtypeStruct((n, f), jnp.float32),
    )(h, parts, W, b.reshape(1, -1), fw.reshape(1, 1))


def _dense_final(h, parts, W2, b2, fw, Wo, bo):
    n, f = h.shape
    c_dim = Wo.shape[1]

    def body(h_ref, p_ref, w2_ref, b2_ref, fw_ref, wo_ref, bo_ref, o_ref):
        hh = h_ref[...]
        t = hh + p_ref[0] + p_ref[1]
        g = jnp.maximum(
            jnp.dot(t, w2_ref[...], preferred_element_type=jnp.float32)
            + b2_ref[...], 0.0) + fw_ref[0, 0] * hh
        logits = jnp.dot(g, wo_ref[...], preferred_element_type=jnp.float32) + bo_ref[...]
        m = jnp.max(logits, axis=-1, keepdims=True)
        lse = jnp.log(jnp.sum(jnp.exp(logits - m), axis=-1, keepdims=True)) + m
        o_ref[...] = logits - lse

    return pl.pallas_call(
        body,
        grid=(n // _ROW_BLOCK,),
        in_specs=[
            pl.BlockSpec((_ROW_BLOCK, f), lambda i: (i, 0)),
            pl.BlockSpec((2, _ROW_BLOCK, f), lambda i: (0, i, 0)),
            pl.BlockSpec((f, f), lambda i: (0, 0)),
            pl.BlockSpec((1, f), lambda i: (0, 0)),
            pl.BlockSpec((1, 1), lambda i: (0, 0)),
            pl.BlockSpec((f, c_dim), lambda i: (0, 0)),
            pl.BlockSpec((1, c_dim), lambda i: (0, 0)),
        ],
        out_specs=pl.BlockSpec((_ROW_BLOCK, c_dim), lambda i: (i, 0)),
        out_shape=jax.ShapeDtypeStruct((n, c_dim), jnp.float32),
    )(h, parts, W2, b2.reshape(1, -1), fw.reshape(1, 1), Wo, bo.reshape(1, -1))


def kernel(x, edge_index, edge_weight, W_first, b_first, W_c1, b_c1, W_c2,
           b_c2, W_out, b_out, fuse_weight):
    n = x.shape[0]
    e = edge_index.shape[1]
    cht = -(-e // (_NS * _K))
    ch0 = max(1, int(round(cht * _SC0_FRAC)))
    ch1 = cht - ch0
    chmax = max(ch0, ch1)
    e_pad = _NS * cht * _K
    n_acc = -(-(n + 1) // (_NS * 8)) * (_NS * 8)

    src = edge_index[0]
    dst = edge_index[1]
    pad = e_pad - e
    src_f = jnp.concatenate([src, jnp.zeros((pad,), src.dtype)])
    dst_f = jnp.concatenate([dst, jnp.full((pad,), n, dst.dtype)])

    def split(a):
        part0 = a[: _NS * ch0 * _K].reshape(_NS, ch0, _K)
        part1 = a[_NS * ch0 * _K:].reshape(_NS, ch1, _K)
        part0 = jnp.pad(part0, ((0, 0), (0, chmax - ch0), (0, 0)))
        part1 = jnp.pad(part1, ((0, 0), (0, chmax - ch1), (0, 0)))
        return jnp.stack([part0, part1])

    src_p, dst_p = split(src_f), split(dst_f)
    zeros = jnp.zeros((n_acc // _NS, _F), jnp.float32)

    h0 = _dense_first(x, W_first, b_first)
    p1 = _segment_sum_sc(h0, src_p, dst_p, zeros, n_acc, ch0, ch1)
    h1 = _dense_mid(h0, p1, W_c1, b_c1, fuse_weight[0])
    p2 = _segment_sum_sc(h1, src_p, dst_p, zeros, n_acc, ch0, ch1)
    return _dense_final(h1, p2, W_c2, b_c2, fuse_weight[1], W_out, b_out)
